# baseline (speedup 1.0000x reference)
import jax
import jax.numpy as jnp
from jax.experimental import pallas as pl
from jax.experimental.pallas import tpu as pltpu

H = 180
W = 180
C = 256
WP = 192
NUM_CLS = 10
OBJ_NUM = 500
R = 30
NB = H // R
NROWS = (R + 2) * WP
LEAD = 3 * WP + 8
FP_ROWS = 187 * WP
TAIL0 = LEAD + H * WP
NCL = 16


def _conv_body(x3_hbm, w9_ref, b_ref, fp_hbm, S, V0, V2, O, Z, sems, osem, zsem):
    i = pl.program_id(0)

    def in_copy(block, slot):
        return pltpu.make_async_copy(
            x3_hbm.at[pl.ds(block * R * WP, NROWS + 16)],
            S.at[slot], sems.at[slot])

    def out_copy(block, slot):
        return pltpu.make_async_copy(
            O.at[slot], fp_hbm.at[pl.ds(block * R * WP + LEAD, R * WP)],
            osem.at[slot])

    @pl.when(i == 0)
    def _():
        in_copy(0, 0).start()
        Z[...] = jnp.zeros((760, C), jnp.float32)
        pltpu.make_async_copy(
            Z.at[pl.ds(0, LEAD)], fp_hbm.at[pl.ds(0, LEAD)], zsem).start()
        pltpu.make_async_copy(
            Z.at[pl.ds(0, LEAD)], fp_hbm.at[pl.ds(0, LEAD)], zsem).wait()
        pltpu.make_async_copy(
            Z.at[pl.ds(0, FP_ROWS - TAIL0)],
            fp_hbm.at[pl.ds(TAIL0, FP_ROWS - TAIL0)], zsem).start()
        pltpu.make_async_copy(
            Z.at[pl.ds(0, FP_ROWS - TAIL0)],
            fp_hbm.at[pl.ds(TAIL0, FP_ROWS - TAIL0)], zsem).wait()

    @pl.when(i + 1 < NB)
    def _():
        in_copy(i + 1, (i + 1) % 2).start()

    in_copy(i, i % 2).wait()
    slot = i % 2
    V0[...] = S[slot, pl.ds(7, NROWS), :]
    V2[...] = S[slot, pl.ds(9, NROWS), :]

    def lhs(dy, dx):
        if dx == 0:
            return V0[pl.ds(dy * WP, R * WP), :]
        if dx == 2:
            return V2[pl.ds(dy * WP, R * WP), :]
        return S[slot, pl.ds(dy * WP + 8, R * WP), :]

    acc = jnp.dot(lhs(0, 0), w9_ref[0], preferred_element_type=jnp.float32)
    for k in range(1, 9):
        dy, dx = divmod(k, 3)
        acc = acc + jnp.dot(lhs(dy, dx), w9_ref[k],
                            preferred_element_type=jnp.float32)
    col = jax.lax.broadcasted_iota(jnp.int32, (R * WP, C), 0) % WP
    valid = col < W

    @pl.when(i >= 2)
    def _():
        out_copy(i - 2, slot).wait()

    O[slot] = jnp.where(valid, jnp.maximum(acc + b_ref[0], 0.0), 0.0)
    out_copy(i, slot).start()

    @pl.when(i == NB - 1)
    def _():
        out_copy(i - 1, (i - 1) % 2).wait()
        out_copy(i, slot).wait()


def _shared_conv(x3, w9, b2):
    return pl.pallas_call(
        _conv_body,
        grid=(NB,),
        in_specs=[
            pl.BlockSpec(memory_space=pltpu.MemorySpace.HBM),
            pl.BlockSpec((9, C, C), lambda i: (0, 0, 0)),
            pl.BlockSpec((1, C), lambda i: (0, 0)),
        ],
        out_specs=pl.BlockSpec(memory_space=pltpu.MemorySpace.HBM),
        out_shape=jax.ShapeDtypeStruct((FP_ROWS, C), jnp.float32),
        scratch_shapes=[
            pltpu.VMEM((2, NROWS + 16, C), jnp.float32),
            pltpu.VMEM((NROWS, C), jnp.float32),
            pltpu.VMEM((NROWS, C), jnp.float32),
            pltpu.VMEM((2, R * WP, C), jnp.float32),
            pltpu.VMEM((760, C), jnp.float32),
            pltpu.SemaphoreType.DMA((2,)),
            pltpu.SemaphoreType.DMA((2,)),
            pltpu.SemaphoreType.DMA,
        ],
    )(x3, w9, b2)


S2ROWS = (R + 6) * WP + 16
HMROWS = (R + 4) * WP
NEG = float('-inf')


def _hm_body(fp_hbm, wc_ref, b_ref, out_ref, S2, sems):
    i = pl.program_id(0)

    def in_copy(block, slot):
        return pltpu.make_async_copy(
            fp_hbm.at[pl.ds(block * R * WP, S2ROWS)],
            S2.at[slot], sems.at[slot])

    @pl.when(i == 0)
    def _():
        in_copy(0, 0).start()

    @pl.when(i + 1 < NB)
    def _():
        in_copy(i + 1, (i + 1) % 2).start()

    in_copy(i, i % 2).wait()
    slot = i % 2

    parts = jnp.dot(S2[slot], wc_ref[...],
                    preferred_element_type=jnp.float32)
    hm = parts[7:7 + HMROWS, 0:NUM_CLS]
    for k in range(1, 9):
        dy, dx = divmod(k, 3)
        off = dy * WP + dx + 7
        hm = hm + parts[off:off + HMROWS, k * 10:k * 10 + NUM_CLS]
    hm = jax.nn.sigmoid(hm + b_ref[0:1, 0:NUM_CLS])
    q = jax.lax.broadcasted_iota(jnp.int32, (HMROWS, NUM_CLS), 0)
    hglob = i * R + q // WP - 2
    valid = (hglob >= 0) & (hglob < H) & (q % WP < W)
    hm = jnp.where(valid, hm, NEG)
    base = 2 * WP
    center = hm[base:base + R * WP, :]
    hmax = center
    for k in range(9):
        dy, dx = divmod(k, 3)
        if (dy, dx) == (1, 1):
            continue
        off = base + (dy - 1) * WP + (dx - 1)
        hmax = jnp.maximum(hmax, hm[off:off + R * WP, :])
    scores = jnp.where(hmax == center, jnp.maximum(center, 0.0), 0.0)
    out_ref[...] = jnp.concatenate(
        [scores, jnp.zeros((R * WP, NCL - NUM_CLS), jnp.float32)], axis=1)


def _hm_nms(feat_pad, wcat, b_pad):
    return pl.pallas_call(
        _hm_body,
        grid=(NB,),
        in_specs=[
            pl.BlockSpec(memory_space=pltpu.MemorySpace.HBM),
            pl.BlockSpec((C, 128), lambda i: (0, 0)),
            pl.BlockSpec((8, NCL), lambda i: (0, 0)),
        ],
        out_specs=pl.BlockSpec((R * WP, NCL), lambda i: (i, 0)),
        out_shape=jax.ShapeDtypeStruct((H * WP, NCL), jnp.float32),
        scratch_shapes=[
            pltpu.VMEM((2, S2ROWS, C), jnp.float32),
            pltpu.SemaphoreType.DMA((2,)),
        ],
    )(feat_pad, wcat, b_pad)


def kernel(x, W_shared, b_shared, W_hm, b_hm):
    xt = jnp.transpose(x[0], (1, 2, 0))
    x3 = jnp.zeros((183, WP, C), jnp.float32)
    x3 = x3.at[1:H + 1, 8:8 + W, :].set(xt).reshape(183 * WP, C)
    w9 = jnp.transpose(W_shared, (2, 3, 1, 0)).reshape(9, C, C)
    feat_pad = _shared_conv(x3, w9, b_shared[None, :])

    wcat = jnp.transpose(W_hm, (1, 2, 3, 0)).reshape(C, 90)
    wcat = jnp.pad(wcat, ((0, 0), (0, 38)))
    b_pad = jnp.zeros((8, NCL), jnp.float32).at[0, :NUM_CLS].set(b_hm)
    scores_wide = _hm_nms(feat_pad, wcat, b_pad)

    flat = scores_wide.reshape(1, H * WP * NCL)
    scores, inds = jax.lax.top_k(flat, OBJ_NUM)
    clses = (inds % NCL).astype(jnp.int32)
    row = inds // NCL
    ys = (row // WP).astype(jnp.float32)
    xs = (row % WP).astype(jnp.float32)
    ct_feat = jnp.take(feat_pad, row[0] + LEAD, axis=0)[None]
    return ct_feat, scores, xs, ys, clses


# speedup vs baseline: 1.7647x; 1.7647x over previous
import functools

import jax
import jax.numpy as jnp
from jax import lax
from jax.experimental import pallas as pl
from jax.experimental.pallas import tpu as pltpu
from jax.experimental.pallas import tpu_sc as plsc

H = 180
W = 180
C = 256
WP = 192
NUM_CLS = 10
OBJ_NUM = 500
R = 30
NB = H // R
NROWS = (R + 2) * WP
LEAD = 3 * WP + 8
FP_ROWS = 187 * WP
TAIL0 = LEAD + H * WP
NCL = 16


def _conv_body(x3_hbm, w9_ref, b_ref, fp_hbm, S, V0, V2, O, Z, sems, osem, zsem):
    i = pl.program_id(0)

    def in_copy(block, slot):
        return pltpu.make_async_copy(
            x3_hbm.at[pl.ds(block * R * WP, NROWS + 16)],
            S.at[slot], sems.at[slot])

    def out_copy(block, slot):
        return pltpu.make_async_copy(
            O.at[slot], fp_hbm.at[pl.ds(block * R * WP + LEAD, R * WP)],
            osem.at[slot])

    @pl.when(i == 0)
    def _():
        in_copy(0, 0).start()
        Z[...] = jnp.zeros((760, C), jnp.float32)
        pltpu.make_async_copy(
            Z.at[pl.ds(0, LEAD)], fp_hbm.at[pl.ds(0, LEAD)], zsem).start()
        pltpu.make_async_copy(
            Z.at[pl.ds(0, LEAD)], fp_hbm.at[pl.ds(0, LEAD)], zsem).wait()
        pltpu.make_async_copy(
            Z.at[pl.ds(0, FP_ROWS - TAIL0)],
            fp_hbm.at[pl.ds(TAIL0, FP_ROWS - TAIL0)], zsem).start()
        pltpu.make_async_copy(
            Z.at[pl.ds(0, FP_ROWS - TAIL0)],
            fp_hbm.at[pl.ds(TAIL0, FP_ROWS - TAIL0)], zsem).wait()

    @pl.when(i + 1 < NB)
    def _():
        in_copy(i + 1, (i + 1) % 2).start()

    in_copy(i, i % 2).wait()
    slot = i % 2
    V0[...] = S[slot, pl.ds(7, NROWS), :]
    V2[...] = S[slot, pl.ds(9, NROWS), :]

    def lhs(dy, dx):
        if dx == 0:
            return V0[pl.ds(dy * WP, R * WP), :]
        if dx == 2:
            return V2[pl.ds(dy * WP, R * WP), :]
        return S[slot, pl.ds(dy * WP + 8, R * WP), :]

    acc = jnp.dot(lhs(0, 0), w9_ref[0], preferred_element_type=jnp.float32)
    for k in range(1, 9):
        dy, dx = divmod(k, 3)
        acc = acc + jnp.dot(lhs(dy, dx), w9_ref[k],
                            preferred_element_type=jnp.float32)
    col = jax.lax.broadcasted_iota(jnp.int32, (R * WP, C), 0) % WP
    valid = col < W

    @pl.when(i >= 2)
    def _():
        out_copy(i - 2, slot).wait()

    O[slot] = jnp.where(valid, jnp.maximum(acc + b_ref[0], 0.0), 0.0)
    out_copy(i, slot).start()

    @pl.when(i == NB - 1)
    def _():
        out_copy(i - 1, (i - 1) % 2).wait()
        out_copy(i, slot).wait()


def _shared_conv(x3, w9, b2):
    return pl.pallas_call(
        _conv_body,
        grid=(NB,),
        in_specs=[
            pl.BlockSpec(memory_space=pltpu.MemorySpace.HBM),
            pl.BlockSpec((9, C, C), lambda i: (0, 0, 0)),
            pl.BlockSpec((1, C), lambda i: (0, 0)),
        ],
        out_specs=pl.BlockSpec(memory_space=pltpu.MemorySpace.HBM),
        out_shape=jax.ShapeDtypeStruct((FP_ROWS, C), jnp.float32),
        scratch_shapes=[
            pltpu.VMEM((2, NROWS + 16, C), jnp.float32),
            pltpu.VMEM((NROWS, C), jnp.float32),
            pltpu.VMEM((NROWS, C), jnp.float32),
            pltpu.VMEM((2, R * WP, C), jnp.float32),
            pltpu.VMEM((760, C), jnp.float32),
            pltpu.SemaphoreType.DMA((2,)),
            pltpu.SemaphoreType.DMA((2,)),
            pltpu.SemaphoreType.DMA,
        ],
    )(x3, w9, b2)


S2ROWS = (R + 6) * WP + 16
HMROWS = (R + 4) * WP
NEG = float('-inf')


def _hm_body(fp_hbm, wc_ref, b_ref, out_ref, S2, sems):
    i = pl.program_id(0)

    def in_copy(block, slot):
        return pltpu.make_async_copy(
            fp_hbm.at[pl.ds(block * R * WP, S2ROWS)],
            S2.at[slot], sems.at[slot])

    @pl.when(i == 0)
    def _():
        in_copy(0, 0).start()

    @pl.when(i + 1 < NB)
    def _():
        in_copy(i + 1, (i + 1) % 2).start()

    in_copy(i, i % 2).wait()
    slot = i % 2

    parts = jnp.dot(S2[slot], wc_ref[...],
                    preferred_element_type=jnp.float32)
    hm = parts[7:7 + HMROWS, 0:NUM_CLS]
    for k in range(1, 9):
        dy, dx = divmod(k, 3)
        off = dy * WP + dx + 7
        hm = hm + parts[off:off + HMROWS, k * 10:k * 10 + NUM_CLS]
    hm = jax.nn.sigmoid(hm + b_ref[0:1, 0:NUM_CLS])
    q = jax.lax.broadcasted_iota(jnp.int32, (HMROWS, NUM_CLS), 0)
    hglob = i * R + q // WP - 2
    valid = (hglob >= 0) & (hglob < H) & (q % WP < W)
    hm = jnp.where(valid, hm, NEG)
    base = 2 * WP
    center = hm[base:base + R * WP, :]
    hmax = center
    for k in range(9):
        dy, dx = divmod(k, 3)
        if (dy, dx) == (1, 1):
            continue
        off = base + (dy - 1) * WP + (dx - 1)
        hmax = jnp.maximum(hmax, hm[off:off + R * WP, :])
    scores = jnp.where(hmax == center, jnp.maximum(center, 0.0), 0.0)
    out_ref[...] = jnp.concatenate(
        [scores, jnp.zeros((R * WP, NCL - NUM_CLS), jnp.float32)], axis=1)


def _hm_nms(feat_pad, wcat, b_pad):
    return pl.pallas_call(
        _hm_body,
        grid=(NB,),
        in_specs=[
            pl.BlockSpec(memory_space=pltpu.MemorySpace.HBM),
            pl.BlockSpec((C, 128), lambda i: (0, 0)),
            pl.BlockSpec((8, NCL), lambda i: (0, 0)),
        ],
        out_specs=pl.BlockSpec((R * WP, NCL), lambda i: (i, 0)),
        out_shape=jax.ShapeDtypeStruct((H * WP, NCL), jnp.float32),
        scratch_shapes=[
            pltpu.VMEM((2, S2ROWS, C), jnp.float32),
            pltpu.SemaphoreType.DMA((2,)),
        ],
    )(feat_pad, wcat, b_pad)


NFLAT = H * WP * NCL
NVREG = NFLAT // 16
HB = 384
HLO = 0.125
HSCALE = HB / 0.875
CCAP = 512
NW = 32
HCHUNK = NVREG // 16
CCHUNK = NVREG // NW


def _sc_topk_body(scores_hbm, candS_hbm, candI_hbm, t_hbm,
                  V, hist, tmp, histg, candS_v, candI_v, tv, shist):
    c = lax.axis_index("c")
    s = lax.axis_index("s")
    wid = s * 2 + c

    pltpu.sync_copy(scores_hbm.at[pl.ds(s * HCHUNK * 16, HCHUNK * 16)],
                    V.at[pl.ds(0, HCHUNK * 16)])

    def zero_hist(j, _):
        z = jnp.zeros((16,), jnp.int32)
        hist[pl.ds(j * 16, 16)] = z
        histg[pl.ds(j * 16, 16)] = z
        return 0
    lax.fori_loop(0, HB, zero_hist, 0)

    lanes = lax.iota(jnp.int32, 16)
    ones = jnp.ones((16,), jnp.int32)

    def hist_step(r, _):
        sv = V[pl.ds(r * 16, 16)]
        b = ((sv - HLO) * HSCALE).astype(jnp.int32)
        b = jnp.clip(b, 0, HB - 1)
        plsc.addupdate_scatter(hist, [b * 16 + lanes], ones)
        return 0
    lax.fori_loop(0, HCHUNK, hist_step, 0)

    pltpu.sync_copy(hist, shist.at[s])
    plsc.subcore_barrier()

    def merge_tile(t, _):
        pltpu.sync_copy(shist.at[t], tmp)

        def add_row(j, _):
            histg[pl.ds(j * 16, 16)] = (histg[pl.ds(j * 16, 16)]
                                        + tmp[pl.ds(j * 16, 16)])
            return 0
        lax.fori_loop(0, HB, add_row, 0)
        return 0
    lax.fori_loop(0, 16, merge_tile, 0)

    def scan_step(t, carry):
        cnt, bsel = carry
        bc = HB - 1 - t
        row = histg[pl.ds(bc * 16, 16)]
        cnt2 = cnt + jnp.sum(row)
        bsel = jnp.where((cnt < OBJ_NUM) & (cnt2 >= OBJ_NUM), bc, bsel)
        return cnt2, bsel
    total, bsel = lax.fori_loop(0, HB, scan_step,
                                (jnp.int32(0), jnp.int32(-1)))
    bsel = jnp.where(bsel < 0, 0, bsel)
    T = jnp.where(bsel > 0,
                  HLO + (bsel - 1).astype(jnp.float32) * (1.0 / HSCALE),
                  1e-38).astype(jnp.float32)

    @pl.when(wid == 0)
    def _():
        tv[...] = jnp.full((16,), T, jnp.float32)
        pltpu.sync_copy(tv, t_hbm)

    pltpu.sync_copy(scores_hbm.at[pl.ds(wid * CCHUNK * 16, CCHUNK * 16)],
                    V.at[pl.ds(0, CCHUNK * 16)])

    def fill_step(j, _):
        candS_v[pl.ds(j * 16, 16)] = jnp.full((16,), -1.0, jnp.float32)
        candI_v[pl.ds(j * 16, 16)] = jnp.zeros((16,), jnp.int32)
        return 0
    lax.fori_loop(0, CCAP // 16, fill_step, 0)

    base0 = wid * CCHUNK * 16

    def collect_step(r, off):
        sv = V[pl.ds(r * 16, 16)]
        mask = (sv >= T) & (sv > 0.0) & (off < CCAP - 16)
        iv = lanes + (base0 + r * 16)
        plsc.store_compressed(candS_v.at[pl.ds(off, 16)], sv, mask=mask)
        plsc.store_compressed(candI_v.at[pl.ds(off, 16)], iv, mask=mask)
        pc = plsc.all_reduce_population_count(mask)
        return off + pc[0]
    lax.fori_loop(0, CCHUNK, collect_step, jnp.int32(0))

    pltpu.sync_copy(candS_v, candS_hbm.at[pl.ds(wid * CCAP, CCAP)])
    pltpu.sync_copy(candI_v, candI_hbm.at[pl.ds(wid * CCAP, CCAP)])


_SC_TOPK_CACHE = []


def _sc_topk(flat):
    if not _SC_TOPK_CACHE:
        _SC_TOPK_CACHE.append(functools.partial(
            pl.kernel,
            out_type=(jax.ShapeDtypeStruct((NW * CCAP,), jnp.float32),
                      jax.ShapeDtypeStruct((NW * CCAP,), jnp.int32),
                      jax.ShapeDtypeStruct((16,), jnp.float32)),
            mesh=plsc.VectorSubcoreMesh(core_axis_name="c",
                                        subcore_axis_name="s"),
            compiler_params=pltpu.CompilerParams(needs_layout_passes=False),
            scratch_types=[
                pltpu.VMEM((HCHUNK * 16,), jnp.float32),
                pltpu.VMEM((HB * 16,), jnp.int32),
                pltpu.VMEM((HB * 16,), jnp.int32),
                pltpu.VMEM((HB * 16,), jnp.int32),
                pltpu.VMEM((CCAP,), jnp.float32),
                pltpu.VMEM((CCAP,), jnp.int32),
                pltpu.VMEM((16,), jnp.float32),
                pltpu.VMEM_SHARED((16, HB * 16), jnp.int32),
            ],
        )(_sc_topk_body))
    return _SC_TOPK_CACHE[0](flat)


def kernel(x, W_shared, b_shared, W_hm, b_hm):
    xt = jnp.transpose(x[0], (1, 2, 0))
    x3 = jnp.zeros((183, WP, C), jnp.float32)
    x3 = x3.at[1:H + 1, 8:8 + W, :].set(xt).reshape(183 * WP, C)
    w9 = jnp.transpose(W_shared, (2, 3, 1, 0)).reshape(9, C, C)
    feat_pad = _shared_conv(x3, w9, b_shared[None, :])

    wcat = jnp.transpose(W_hm, (1, 2, 3, 0)).reshape(C, 90)
    wcat = jnp.pad(wcat, ((0, 0), (0, 38)))
    b_pad = jnp.zeros((8, NCL), jnp.float32).at[0, :NUM_CLS].set(b_hm)
    scores_wide = _hm_nms(feat_pad, wcat, b_pad)

    flat = scores_wide.reshape(H * WP * NCL)
    candS, candI, tvec = _sc_topk(flat)
    s512 = flat[:512]
    extraS = jnp.where((s512 >= tvec[0]) & (s512 > 0.0), -2.0, s512)
    extraI = jnp.arange(512, dtype=jnp.int32)
    allS = jnp.concatenate([candS, extraS])
    allI = jnp.concatenate([candI, extraI])
    scores, sel = jax.lax.top_k(allS[None], OBJ_NUM)
    inds = allI[sel[0]][None]
    clses = (inds % NCL).astype(jnp.int32)
    row = inds // NCL
    ys = (row // WP).astype(jnp.float32)
    xs = (row % WP).astype(jnp.float32)
    ct_feat = jnp.take(feat_pad, row[0] + LEAD, axis=0)[None]
    return ct_feat, scores, xs, ys, clses
